# baseline (speedup 1.0000x reference)
import functools

import jax
import jax.numpy as jnp
from jax import lax
from jax.experimental import pallas as pl
from jax.experimental.pallas import tpu as pltpu
from jax.experimental.pallas import tpu_sc as plsc

NC = 2
NS = 16
LN = 16

N = 10000
E = 160000
NP = 10240
EP = 163840
C = 128
D = 256
HD = 256
HH = 128
G = 64
L = 64

_MESH = plsc.VectorSubcoreMesh(core_axis_name="c", subcore_axis_name="s")


NCHK = EP // 128 // (NC * NS)


@functools.partial(
    pl.kernel,
    mesh=_MESH,
    out_type=jax.ShapeDtypeStruct((2 * NP, 16), jnp.float32),
    scratch_types=[
        pltpu.VMEM_SHARED((NP, 16), jnp.float32),
        pltpu.VMEM((C,), jnp.int32),
        pltpu.VMEM((C,), jnp.int32),
        pltpu.VMEM((C, 16), jnp.float32),
        pltpu.VMEM((C, 16), jnp.float32),
        pltpu.SemaphoreType.DMA,
        pltpu.SemaphoreType.DMA,
    ],
)
def _deg_sc(dst_hbm, initc_hbm, deg_hbm, acc_sh, dv0, dv1, ones_v, ini_v,
            sem_i0, sem_i1):
    cid = lax.axis_index("c")
    sid = lax.axis_index("s")
    wid = cid * NS + sid
    base = wid * NCHK * C
    dvs = (dv0, dv1)
    sems = (sem_i0, sem_i1)

    def i_start(k, p):
        pltpu.async_copy(dst_hbm.at[pl.ds(base + k * C, C)], dvs[p], sems[p])

    def i_wait(k, p):
        pltpu.make_async_copy(dst_hbm.at[pl.ds(base + k * C, C)],
                              dvs[p], sems[p]).wait()

    pltpu.sync_copy(initc_hbm.at[pl.ds(0, C)], ones_v)
    pltpu.sync_copy(initc_hbm.at[pl.ds(cid * C, C)], ini_v)
    for k in range(NP // NS // C):
        pltpu.sync_copy(ini_v, acc_sh.at[pl.ds(sid * (NP // NS) + k * C, C)])
    plsc.subcore_barrier()
    i_start(0, 0)
    i_start(1, 1)

    def body(h, _):
        for j in range(2):
            k = 2 * h + j
            i_wait(k, j)
            pltpu.sync_copy(ones_v, acc_sh.at[dvs[j]], add=True)

            @pl.when(k + 2 < NCHK)
            def _():
                i_start(k + 2, j)
        return 0

    lax.fori_loop(0, NCHK // 2, body, 0)
    plsc.subcore_barrier()
    for k in range(NP // NS // C):
        off = sid * (NP // NS) + k * C
        pltpu.sync_copy(acc_sh.at[pl.ds(off, C)], ini_v)
        pltpu.sync_copy(ini_v, deg_hbm.at[pl.ds(cid * NP + off, C)])


KT = EP // 128 // NS


@functools.partial(
    pl.kernel,
    mesh=_MESH,
    out_type=jax.ShapeDtypeStruct((2 * NP, HH), jnp.float32),
    scratch_types=[
        pltpu.VMEM_SHARED((NP, HH), jnp.float32),
        pltpu.VMEM((C,), jnp.int32),
        pltpu.VMEM((C,), jnp.int32),
        pltpu.VMEM((C, HH), jnp.float32),
        pltpu.SemaphoreType.DMA,
    ],
)
def _spmm_sc(y_hbm, src_hbm, dst_hbm, out_hbm, acc_sh, src_v, dst_v,
             rows_v, sem):
    cid = lax.axis_index("c")
    sid = lax.axis_index("s")
    row0 = cid * NP

    for k in range(NP // NS // C):
        off = sid * (NP // NS) + k * C
        pltpu.sync_copy(y_hbm.at[pl.ds(row0 + off, C)], rows_v)
        pltpu.sync_copy(rows_v, acc_sh.at[pl.ds(off, C)])
    plsc.subcore_barrier()

    base = sid * KT * C

    def body(i, _):
        pltpu.sync_copy(src_hbm.at[pl.ds(base + i * C, C)], src_v)
        pltpu.sync_copy(dst_hbm.at[pl.ds(base + i * C, C)], dst_v)
        for j in range(C // LN):
            sl = pl.ds(j * LN, LN)
            src_v[sl] = src_v[sl] + row0
        pltpu.async_copy(y_hbm.at[src_v], rows_v, sem).wait()
        pltpu.sync_copy(rows_v, acc_sh.at[dst_v], add=True)
        return 0

    lax.fori_loop(0, KT, body, 0)
    plsc.subcore_barrier()

    for k in range(NP // NS // C):
        off = sid * (NP // NS) + k * C
        pltpu.sync_copy(acc_sh.at[pl.ds(off, C)], rows_v)
        pltpu.sync_copy(rows_v, out_hbm.at[pl.ds(row0 + off, C)])


BM = 512


def _mm1_body(x_ref, w_ref, deg_ref, o_ref):
    dis = lax.rsqrt(deg_ref[0, :, :1] + deg_ref[1, :, :1])
    o_ref[...] = jnp.dot(x_ref[...], w_ref[...],
                         preferred_element_type=jnp.float32) * dis


def _mm1(x_pad, W1, deg):
    return pl.pallas_call(
        _mm1_body,
        grid=(2, NP // BM),
        in_specs=[
            pl.BlockSpec((BM, D), lambda c, i: (i, 0)),
            pl.BlockSpec((D, HH), lambda c, i: (0, c)),
            pl.BlockSpec((2, BM, 16), lambda c, i: (0, i, 0)),
        ],
        out_specs=pl.BlockSpec((BM, HH), lambda c, i: (c * (NP // BM) + i, 0)),
        out_shape=jax.ShapeDtypeStruct((2 * NP, HH), jnp.float32),
    )(x_pad, W1, deg)


def _mm2_body(a_ref, w_ref, deg_ref, b_ref, o_ref):
    dis = lax.rsqrt(deg_ref[0, :, :1] + deg_ref[1, :, :1])
    h = jnp.concatenate([a_ref[0], a_ref[1]], axis=1)
    h = jax.nn.relu(h * dis + b_ref[...])
    y = jnp.dot(h, w_ref[...], preferred_element_type=jnp.float32) * dis
    o_ref[0] = y[:, :HH]
    o_ref[1] = y[:, HH:]


def _mm2(acc3, W2, deg, b1):
    return pl.pallas_call(
        _mm2_body,
        grid=(NP // BM,),
        in_specs=[
            pl.BlockSpec((2, BM, HH), lambda i: (0, i, 0)),
            pl.BlockSpec((HD, HD), lambda i: (0, 0)),
            pl.BlockSpec((2, BM, 16), lambda i: (0, i, 0)),
            pl.BlockSpec((1, HD), lambda i: (0, 0)),
        ],
        out_specs=pl.BlockSpec((2, BM, HH), lambda i: (0, i, 0)),
        out_shape=jax.ShapeDtypeStruct((2, NP, HH), jnp.float32),
    )(acc3, W2, deg, b1)


def _final_body(a_ref, deg_ref, b_ref, bt_ref, wmu_ref, bmu_ref, wlv_ref,
                blv_ref, mu_ref, lv_ref, sums, counts):
    i = pl.program_id(0)

    @pl.when(i == 0)
    def _():
        sums[...] = jnp.zeros_like(sums)
        counts[...] = jnp.zeros_like(counts)

    dis = lax.rsqrt(deg_ref[0, :, :1] + deg_ref[1, :, :1])
    h = jnp.concatenate([a_ref[0], a_ref[1]], axis=1)
    h = jax.nn.relu(h * dis + b_ref[...])
    ids = lax.broadcasted_iota(jnp.int32, (G, BM), 0)
    oh = (ids == jnp.reshape(bt_ref[...], (1, BM))).astype(jnp.float32)
    sums[...] += jnp.dot(oh, h, preferred_element_type=jnp.float32)
    counts[...] += jnp.broadcast_to(
        jnp.sum(oh, axis=1, keepdims=True), counts.shape)

    @pl.when(i == pl.num_programs(0) - 1)
    def _():
        hg = sums[...] / jnp.maximum(counts[:, :1], 1.0)
        mu_ref[...] = jnp.dot(hg, wmu_ref[...],
                              preferred_element_type=jnp.float32) + bmu_ref[...]
        lv_ref[...] = jnp.dot(hg, wlv_ref[...],
                              preferred_element_type=jnp.float32) + blv_ref[...]


def _final(acc3, deg, b2, batch2d, Wmu, bmu, Wlv, blv):
    return pl.pallas_call(
        _final_body,
        grid=(NP // BM,),
        in_specs=[
            pl.BlockSpec((2, BM, HH), lambda i: (0, i, 0)),
            pl.BlockSpec((2, BM, 16), lambda i: (0, i, 0)),
            pl.BlockSpec((1, HD), lambda i: (0, 0)),
            pl.BlockSpec((BM, 1), lambda i: (i, 0)),
            pl.BlockSpec((HD, L), lambda i: (0, 0)),
            pl.BlockSpec((1, L), lambda i: (0, 0)),
            pl.BlockSpec((HD, L), lambda i: (0, 0)),
            pl.BlockSpec((1, L), lambda i: (0, 0)),
        ],
        out_specs=[
            pl.BlockSpec((G, L), lambda i: (0, 0)),
            pl.BlockSpec((G, L), lambda i: (0, 0)),
        ],
        out_shape=[
            jax.ShapeDtypeStruct((G, L), jnp.float32),
            jax.ShapeDtypeStruct((G, L), jnp.float32),
        ],
        scratch_shapes=[
            pltpu.VMEM((G, HD), jnp.float32),
            pltpu.VMEM((G, 128), jnp.float32),
        ],
    )(acc3, deg, b2, batch2d, Wmu, bmu, Wlv, blv)


def kernel(x, edge_index, batch, W1, b1, W2, b2, Wmu, bmu, Wlv, blv):
    x_pad = jnp.zeros((NP, D), jnp.float32).at[:N].set(x)
    padi = jnp.full((EP - E,), N, jnp.int32)
    srcp = jnp.concatenate([edge_index[0], padi])
    dstp = jnp.concatenate([edge_index[1], padi])
    batch2d = jnp.concatenate(
        [batch, jnp.full((NP - N,), G, jnp.int32)])[:, None]
    initc = jnp.concatenate([jnp.ones((C, 16), jnp.float32),
                             jnp.zeros((C, 16), jnp.float32)])
    b1r = b1[None, :]
    b2r = b2[None, :]
    bmur = bmu[None, :]
    blvr = blv[None, :]

    deg = _deg_sc(dstp, initc).reshape(2, NP, 16)
    y1 = _mm1(x_pad, W1, deg)
    acc1 = _spmm_sc(y1, srcp, dstp)
    y2 = _mm2(acc1.reshape(2, NP, HH), W2, deg, b1r)
    acc2 = _spmm_sc(y2.reshape(2 * NP, HH), srcp, dstp)
    mu, lv = _final(acc2.reshape(2, NP, HH), deg, b2r, batch2d,
                    Wmu, bmur, Wlv, blvr)
    return (mu, lv)


# speedup vs baseline: 1.2595x; 1.2595x over previous
import functools

import jax
import jax.numpy as jnp
from jax import lax
from jax.experimental import pallas as pl
from jax.experimental.pallas import tpu as pltpu
from jax.experimental.pallas import tpu_sc as plsc

NC = 2
NS = 16
LN = 16

N = 10000
E = 160000
NP = 10240
EP = 163840
C = 128
D = 256
HD = 256
HH = 128
G = 64
L = 64

_MESH = plsc.VectorSubcoreMesh(core_axis_name="c", subcore_axis_name="s")


NCHK = EP // 128 // (NC * NS)


@functools.partial(
    pl.kernel,
    mesh=_MESH,
    out_type=jax.ShapeDtypeStruct((2 * NP, 16), jnp.float32),
    scratch_types=[
        pltpu.VMEM_SHARED((NP, 16), jnp.float32),
        pltpu.VMEM((C,), jnp.int32),
        pltpu.VMEM((C,), jnp.int32),
        pltpu.VMEM((C, 16), jnp.float32),
        pltpu.VMEM((C, 16), jnp.float32),
        pltpu.SemaphoreType.DMA,
        pltpu.SemaphoreType.DMA,
    ],
)
def _deg_sc(dst_hbm, initc_hbm, deg_hbm, acc_sh, dv0, dv1, ones_v, ini_v,
            sem_i0, sem_i1):
    cid = lax.axis_index("c")
    sid = lax.axis_index("s")
    wid = cid * NS + sid
    base = wid * NCHK * C
    dvs = (dv0, dv1)
    sems = (sem_i0, sem_i1)

    def i_start(k, p):
        pltpu.async_copy(dst_hbm.at[pl.ds(base + k * C, C)], dvs[p], sems[p])

    def i_wait(k, p):
        pltpu.make_async_copy(dst_hbm.at[pl.ds(base + k * C, C)],
                              dvs[p], sems[p]).wait()

    pltpu.sync_copy(initc_hbm.at[pl.ds(0, C)], ones_v)
    pltpu.sync_copy(initc_hbm.at[pl.ds(cid * C, C)], ini_v)
    for k in range(NP // NS // C):
        pltpu.sync_copy(ini_v, acc_sh.at[pl.ds(sid * (NP // NS) + k * C, C)])
    plsc.subcore_barrier()
    i_start(0, 0)
    i_start(1, 1)

    def body(h, _):
        for j in range(2):
            k = 2 * h + j
            i_wait(k, j)
            pltpu.sync_copy(ones_v, acc_sh.at[dvs[j]], add=True)

            @pl.when(k + 2 < NCHK)
            def _():
                i_start(k + 2, j)
        return 0

    lax.fori_loop(0, NCHK // 2, body, 0)
    plsc.subcore_barrier()
    for k in range(NP // NS // C):
        off = sid * (NP // NS) + k * C
        pltpu.sync_copy(acc_sh.at[pl.ds(off, C)], ini_v)
        pltpu.sync_copy(ini_v, deg_hbm.at[pl.ds(cid * NP + off, C)])


KT = EP // 128 // NS


@functools.partial(
    pl.kernel,
    mesh=_MESH,
    out_type=jax.ShapeDtypeStruct((2 * NP, HH), jnp.float32),
    scratch_types=[
        pltpu.VMEM_SHARED((NP, HH), jnp.float32),
        pltpu.VMEM((C,), jnp.int32),
        pltpu.VMEM((C,), jnp.int32),
        pltpu.VMEM((C,), jnp.int32),
        pltpu.VMEM((C,), jnp.int32),
        pltpu.VMEM((C,), jnp.int32),
        pltpu.VMEM((C,), jnp.int32),
        pltpu.VMEM((C,), jnp.int32),
        pltpu.VMEM((C,), jnp.int32),
        pltpu.VMEM((C, HH), jnp.float32),
        pltpu.VMEM((C, HH), jnp.float32),
        pltpu.SemaphoreType.DMA,
        pltpu.SemaphoreType.DMA,
        pltpu.SemaphoreType.DMA,
        pltpu.SemaphoreType.DMA,
        pltpu.SemaphoreType.DMA,
        pltpu.SemaphoreType.DMA,
        pltpu.SemaphoreType.DMA,
    ],
)
def _spmm_sc(y_hbm, src_hbm, dst_hbm, out_hbm, acc_sh,
             sv0, sv1, sv2, sv3, dv0, dv1, dv2, dv3, rows0, rows1,
             sem_i0, sem_i1, sem_i2, sem_i3, sem_g0, sem_g1, sem_s):
    cid = lax.axis_index("c")
    sid = lax.axis_index("s")
    row0 = cid * NP
    base = sid * KT * C
    svs = (sv0, sv1, sv2, sv3)
    dvs = (dv0, dv1, dv2, dv3)
    isems = (sem_i0, sem_i1, sem_i2, sem_i3)
    rows = (rows0, rows1)
    gsems = (sem_g0, sem_g1)

    def i_start(k, p):
        pltpu.async_copy(src_hbm.at[pl.ds(base + k * C, C)], svs[p], isems[p])
        pltpu.async_copy(dst_hbm.at[pl.ds(base + k * C, C)], dvs[p], isems[p])

    def i_wait(k, p):
        pltpu.make_async_copy(src_hbm.at[pl.ds(base + k * C, C)],
                              svs[p], isems[p]).wait()
        pltpu.make_async_copy(dst_hbm.at[pl.ds(base + k * C, C)],
                              dvs[p], isems[p]).wait()
        for j in range(C // LN):
            sl = pl.ds(j * LN, LN)
            svs[p][sl] = svs[p][sl] + row0

    def g_start(p, rp):
        pltpu.async_copy(y_hbm.at[svs[p]], rows[rp], gsems[rp])

    def g_wait(p, rp):
        pltpu.make_async_copy(y_hbm.at[svs[p]], rows[rp], gsems[rp]).wait()

    def s_start(p, rp):
        pltpu.async_copy(rows[rp], acc_sh.at[dvs[p]], sem_s, add=True)

    def s_wait(p, rp):
        pltpu.make_async_copy(rows[rp], acc_sh.at[dvs[p]], sem_s).wait()

    for k in range(NP // NS // C):
        off = sid * (NP // NS) + k * C
        pltpu.sync_copy(y_hbm.at[pl.ds(row0 + off, C)], rows0)
        pltpu.sync_copy(rows0, acc_sh.at[pl.ds(off, C)])
    plsc.subcore_barrier()

    i_start(0, 0)
    i_start(1, 1)
    i_start(2, 2)
    i_wait(0, 0)
    g_start(0, 0)

    def body(h, _):
        for j in range(4):
            k = 4 * h + j
            rp = j % 2
            g_wait(j, rp)

            @pl.when(k >= 1)
            def _():
                s_wait((j + 3) % 4, 1 - rp)

            @pl.when(k + 1 < KT)
            def _():
                i_wait(k + 1, (j + 1) % 4)
                g_start((j + 1) % 4, 1 - rp)
            s_start(j, rp)

            @pl.when(k + 3 < KT)
            def _():
                i_start(k + 3, (j + 3) % 4)
        return 0

    lax.fori_loop(0, KT // 4, body, 0)
    s_wait(3, 1)
    plsc.subcore_barrier()

    for k in range(NP // NS // C):
        off = sid * (NP // NS) + k * C
        pltpu.sync_copy(acc_sh.at[pl.ds(off, C)], rows0)
        pltpu.sync_copy(rows0, out_hbm.at[pl.ds(row0 + off, C)])


BM = 512


def _mm1_body(x_ref, w_ref, deg_ref, o_ref):
    dis = lax.rsqrt(deg_ref[0, :, :1] + deg_ref[1, :, :1])
    o_ref[...] = jnp.dot(x_ref[...], w_ref[...],
                         preferred_element_type=jnp.float32) * dis


def _mm1(x_pad, W1, deg):
    return pl.pallas_call(
        _mm1_body,
        grid=(2, NP // BM),
        in_specs=[
            pl.BlockSpec((BM, D), lambda c, i: (i, 0)),
            pl.BlockSpec((D, HH), lambda c, i: (0, c)),
            pl.BlockSpec((2, BM, 16), lambda c, i: (0, i, 0)),
        ],
        out_specs=pl.BlockSpec((BM, HH), lambda c, i: (c * (NP // BM) + i, 0)),
        out_shape=jax.ShapeDtypeStruct((2 * NP, HH), jnp.float32),
    )(x_pad, W1, deg)


def _mm2_body(a_ref, w_ref, deg_ref, b_ref, o_ref):
    dis = lax.rsqrt(deg_ref[0, :, :1] + deg_ref[1, :, :1])
    h = jnp.concatenate([a_ref[0], a_ref[1]], axis=1)
    h = jax.nn.relu(h * dis + b_ref[...])
    y = jnp.dot(h, w_ref[...], preferred_element_type=jnp.float32) * dis
    o_ref[0] = y[:, :HH]
    o_ref[1] = y[:, HH:]


def _mm2(acc3, W2, deg, b1):
    return pl.pallas_call(
        _mm2_body,
        grid=(NP // BM,),
        in_specs=[
            pl.BlockSpec((2, BM, HH), lambda i: (0, i, 0)),
            pl.BlockSpec((HD, HD), lambda i: (0, 0)),
            pl.BlockSpec((2, BM, 16), lambda i: (0, i, 0)),
            pl.BlockSpec((1, HD), lambda i: (0, 0)),
        ],
        out_specs=pl.BlockSpec((2, BM, HH), lambda i: (0, i, 0)),
        out_shape=jax.ShapeDtypeStruct((2, NP, HH), jnp.float32),
    )(acc3, W2, deg, b1)


def _final_body(a_ref, deg_ref, b_ref, bt_ref, wmu_ref, bmu_ref, wlv_ref,
                blv_ref, mu_ref, lv_ref, sums, counts):
    i = pl.program_id(0)

    @pl.when(i == 0)
    def _():
        sums[...] = jnp.zeros_like(sums)
        counts[...] = jnp.zeros_like(counts)

    dis = lax.rsqrt(deg_ref[0, :, :1] + deg_ref[1, :, :1])
    h = jnp.concatenate([a_ref[0], a_ref[1]], axis=1)
    h = jax.nn.relu(h * dis + b_ref[...])
    ids = lax.broadcasted_iota(jnp.int32, (G, BM), 0)
    oh = (ids == jnp.reshape(bt_ref[...], (1, BM))).astype(jnp.float32)
    sums[...] += jnp.dot(oh, h, preferred_element_type=jnp.float32)
    counts[...] += jnp.broadcast_to(
        jnp.sum(oh, axis=1, keepdims=True), counts.shape)

    @pl.when(i == pl.num_programs(0) - 1)
    def _():
        hg = sums[...] / jnp.maximum(counts[:, :1], 1.0)
        mu_ref[...] = jnp.dot(hg, wmu_ref[...],
                              preferred_element_type=jnp.float32) + bmu_ref[...]
        lv_ref[...] = jnp.dot(hg, wlv_ref[...],
                              preferred_element_type=jnp.float32) + blv_ref[...]


def _final(acc3, deg, b2, batch2d, Wmu, bmu, Wlv, blv):
    return pl.pallas_call(
        _final_body,
        grid=(NP // BM,),
        in_specs=[
            pl.BlockSpec((2, BM, HH), lambda i: (0, i, 0)),
            pl.BlockSpec((2, BM, 16), lambda i: (0, i, 0)),
            pl.BlockSpec((1, HD), lambda i: (0, 0)),
            pl.BlockSpec((BM, 1), lambda i: (i, 0)),
            pl.BlockSpec((HD, L), lambda i: (0, 0)),
            pl.BlockSpec((1, L), lambda i: (0, 0)),
            pl.BlockSpec((HD, L), lambda i: (0, 0)),
            pl.BlockSpec((1, L), lambda i: (0, 0)),
        ],
        out_specs=[
            pl.BlockSpec((G, L), lambda i: (0, 0)),
            pl.BlockSpec((G, L), lambda i: (0, 0)),
        ],
        out_shape=[
            jax.ShapeDtypeStruct((G, L), jnp.float32),
            jax.ShapeDtypeStruct((G, L), jnp.float32),
        ],
        scratch_shapes=[
            pltpu.VMEM((G, HD), jnp.float32),
            pltpu.VMEM((G, 128), jnp.float32),
        ],
    )(acc3, deg, b2, batch2d, Wmu, bmu, Wlv, blv)


def kernel(x, edge_index, batch, W1, b1, W2, b2, Wmu, bmu, Wlv, blv):
    x_pad = jnp.zeros((NP, D), jnp.float32).at[:N].set(x)
    padi = jnp.full((EP - E,), N, jnp.int32)
    srcp = jnp.concatenate([edge_index[0], padi])
    dstp = jnp.concatenate([edge_index[1], padi])
    batch2d = jnp.concatenate(
        [batch, jnp.full((NP - N,), G, jnp.int32)])[:, None]
    initc = jnp.concatenate([jnp.ones((C, 16), jnp.float32),
                             jnp.zeros((C, 16), jnp.float32)])
    b1r = b1[None, :]
    b2r = b2[None, :]
    bmur = bmu[None, :]
    blvr = blv[None, :]

    deg = _deg_sc(dstp, initc).reshape(2, NP, 16)
    y1 = _mm1(x_pad, W1, deg)
    acc1 = _spmm_sc(y1, srcp, dstp)
    y2 = _mm2(acc1.reshape(2, NP, HH), W2, deg, b1r)
    acc2 = _spmm_sc(y2.reshape(2 * NP, HH), srcp, dstp)
    mu, lv = _final(acc2.reshape(2, NP, HH), deg, b2r, batch2d,
                    Wmu, bmur, Wlv, blvr)
    return (mu, lv)


# speedup vs baseline: 1.2876x; 1.0223x over previous
import functools

import jax
import jax.numpy as jnp
from jax import lax
from jax.experimental import pallas as pl
from jax.experimental.pallas import tpu as pltpu
from jax.experimental.pallas import tpu_sc as plsc

NC = 2
NS = 16
LN = 16

N = 10000
E = 160000
NP = 10240
EP = 163840
C = 128
D = 256
HD = 256
HH = 128
G = 64
L = 64

_MESH = plsc.VectorSubcoreMesh(core_axis_name="c", subcore_axis_name="s")


NCHK = EP // 128 // (NC * NS)


@functools.partial(
    pl.kernel,
    mesh=_MESH,
    out_type=jax.ShapeDtypeStruct((2 * NP, 16), jnp.float32),
    scratch_types=[
        pltpu.VMEM_SHARED((NP, 16), jnp.float32),
        pltpu.VMEM((C,), jnp.int32),
        pltpu.VMEM((C,), jnp.int32),
        pltpu.VMEM((C, 16), jnp.float32),
        pltpu.VMEM((C, 16), jnp.float32),
        pltpu.SemaphoreType.DMA,
        pltpu.SemaphoreType.DMA,
    ],
)
def _deg_sc(dst_hbm, initc_hbm, deg_hbm, acc_sh, dv0, dv1, ones_v, ini_v,
            sem_i0, sem_i1):
    cid = lax.axis_index("c")
    sid = lax.axis_index("s")
    wid = cid * NS + sid
    base = wid * NCHK * C
    dvs = (dv0, dv1)
    sems = (sem_i0, sem_i1)

    def i_start(k, p):
        pltpu.async_copy(dst_hbm.at[pl.ds(base + k * C, C)], dvs[p], sems[p])

    def i_wait(k, p):
        pltpu.make_async_copy(dst_hbm.at[pl.ds(base + k * C, C)],
                              dvs[p], sems[p]).wait()

    pltpu.sync_copy(initc_hbm.at[pl.ds(0, C)], ones_v)
    pltpu.sync_copy(initc_hbm.at[pl.ds(cid * C, C)], ini_v)
    for k in range(NP // NS // C):
        pltpu.sync_copy(ini_v, acc_sh.at[pl.ds(sid * (NP // NS) + k * C, C)])
    plsc.subcore_barrier()
    i_start(0, 0)
    i_start(1, 1)

    def body(h, _):
        for j in range(2):
            k = 2 * h + j
            i_wait(k, j)
            pltpu.sync_copy(ones_v, acc_sh.at[dvs[j]], add=True)

            @pl.when(k + 2 < NCHK)
            def _():
                i_start(k + 2, j)
        return 0

    lax.fori_loop(0, NCHK // 2, body, 0)
    plsc.subcore_barrier()
    for k in range(NP // NS // C):
        off = sid * (NP // NS) + k * C
        pltpu.sync_copy(acc_sh.at[pl.ds(off, C)], ini_v)
        pltpu.sync_copy(ini_v, deg_hbm.at[pl.ds(cid * NP + off, C)])


CS = 64
KT = EP // CS // NS
CW = 64


@functools.partial(
    pl.kernel,
    mesh=_MESH,
    out_type=jax.ShapeDtypeStruct((2 * NP, HH), jnp.float32),
    scratch_types=(
        [pltpu.VMEM_SHARED((NP, HH), jnp.float32)]
        + [pltpu.VMEM((CS,), jnp.int32)] * 16
        + [pltpu.VMEM((CS, HH), jnp.float32)] * 4
        + [pltpu.VMEM((CW, HH), jnp.float32)]
        + [pltpu.SemaphoreType.DMA] * 14
    ),
)
def _spmm_sc(y_hbm, src_hbm, dst_hbm, out_hbm, acc_sh, *bufs):
    svs = bufs[0:8]
    dvs = bufs[8:16]
    rows = bufs[16:20]
    wbuf = bufs[20]
    isems = bufs[21:29]
    gsems = bufs[29:33]
    ssems = bufs[33:35]
    cid = lax.axis_index("c")
    sid = lax.axis_index("s")
    row0 = cid * NP
    base = sid * KT * CS

    def i_start(k, p):
        pltpu.async_copy(src_hbm.at[pl.ds(base + k * CS, CS)], svs[p], isems[p])
        pltpu.async_copy(dst_hbm.at[pl.ds(base + k * CS, CS)], dvs[p], isems[p])

    def i_wait(k, p):
        pltpu.make_async_copy(src_hbm.at[pl.ds(base + k * CS, CS)],
                              svs[p], isems[p]).wait()
        pltpu.make_async_copy(dst_hbm.at[pl.ds(base + k * CS, CS)],
                              dvs[p], isems[p]).wait()
        for j in range(CS // LN):
            sl = pl.ds(j * LN, LN)
            svs[p][sl] = svs[p][sl] + row0

    def g_start(p, rp):
        pltpu.async_copy(y_hbm.at[svs[p]], rows[rp], gsems[rp])

    def g_wait(p, rp):
        pltpu.make_async_copy(y_hbm.at[svs[p]], rows[rp], gsems[rp]).wait()

    def s_start(p, rp, sp):
        pltpu.async_copy(rows[rp], acc_sh.at[dvs[p]], ssems[sp], add=True)

    def s_wait(p, rp, sp):
        pltpu.make_async_copy(rows[rp], acc_sh.at[dvs[p]], ssems[sp]).wait()

    for k in range(NP // NS // CW):
        off = sid * (NP // NS) + k * CW
        pltpu.sync_copy(y_hbm.at[pl.ds(row0 + off, CW)], wbuf)
        pltpu.sync_copy(wbuf, acc_sh.at[pl.ds(off, CW)])
    plsc.subcore_barrier()

    i_start(0, 0)
    i_start(1, 1)
    i_start(2, 2)
    i_start(3, 3)
    i_wait(0, 0)
    g_start(0, 0)
    i_wait(1, 1)
    g_start(1, 1)

    def body(h, _):
        for j in range(8):
            k = 8 * h + j
            jr = j % 4
            g_wait(j, jr)

            @pl.when(k >= 2)
            def _():
                s_wait((j + 6) % 8, (j + 2) % 4, j % 2)

            @pl.when(k + 2 < KT)
            def _():
                i_wait(k + 2, (j + 2) % 8)
                g_start((j + 2) % 8, (j + 2) % 4)
            s_start(j, jr, j % 2)

            @pl.when(k + 4 < KT)
            def _():
                i_start(k + 4, (j + 4) % 8)
        return 0

    lax.fori_loop(0, KT // 8, body, 0)
    s_wait(6, 2, 0)
    s_wait(7, 3, 1)
    plsc.subcore_barrier()

    for k in range(NP // NS // CW):
        off = sid * (NP // NS) + k * CW
        pltpu.sync_copy(acc_sh.at[pl.ds(off, CW)], wbuf)
        pltpu.sync_copy(wbuf, out_hbm.at[pl.ds(row0 + off, CW)])


BM = 512


def _mm1_body(x_ref, w_ref, deg_ref, o_ref):
    dis = lax.rsqrt(deg_ref[0, :, :1] + deg_ref[1, :, :1])
    o_ref[...] = jnp.dot(x_ref[...], w_ref[...],
                         preferred_element_type=jnp.float32) * dis


def _mm1(x_pad, W1, deg):
    return pl.pallas_call(
        _mm1_body,
        grid=(2, NP // BM),
        in_specs=[
            pl.BlockSpec((BM, D), lambda c, i: (i, 0)),
            pl.BlockSpec((D, HH), lambda c, i: (0, c)),
            pl.BlockSpec((2, BM, 16), lambda c, i: (0, i, 0)),
        ],
        out_specs=pl.BlockSpec((BM, HH), lambda c, i: (c * (NP // BM) + i, 0)),
        out_shape=jax.ShapeDtypeStruct((2 * NP, HH), jnp.float32),
    )(x_pad, W1, deg)


def _mm2_body(a_ref, w_ref, deg_ref, b_ref, o_ref):
    dis = lax.rsqrt(deg_ref[0, :, :1] + deg_ref[1, :, :1])
    h = jnp.concatenate([a_ref[0], a_ref[1]], axis=1)
    h = jax.nn.relu(h * dis + b_ref[...])
    y = jnp.dot(h, w_ref[...], preferred_element_type=jnp.float32) * dis
    o_ref[0] = y[:, :HH]
    o_ref[1] = y[:, HH:]


def _mm2(acc3, W2, deg, b1):
    return pl.pallas_call(
        _mm2_body,
        grid=(NP // BM,),
        in_specs=[
            pl.BlockSpec((2, BM, HH), lambda i: (0, i, 0)),
            pl.BlockSpec((HD, HD), lambda i: (0, 0)),
            pl.BlockSpec((2, BM, 16), lambda i: (0, i, 0)),
            pl.BlockSpec((1, HD), lambda i: (0, 0)),
        ],
        out_specs=pl.BlockSpec((2, BM, HH), lambda i: (0, i, 0)),
        out_shape=jax.ShapeDtypeStruct((2, NP, HH), jnp.float32),
    )(acc3, W2, deg, b1)


def _final_body(a_ref, deg_ref, b_ref, bt_ref, wmu_ref, bmu_ref, wlv_ref,
                blv_ref, mu_ref, lv_ref, sums, counts):
    i = pl.program_id(0)

    @pl.when(i == 0)
    def _():
        sums[...] = jnp.zeros_like(sums)
        counts[...] = jnp.zeros_like(counts)

    dis = lax.rsqrt(deg_ref[0, :, :1] + deg_ref[1, :, :1])
    h = jnp.concatenate([a_ref[0], a_ref[1]], axis=1)
    h = jax.nn.relu(h * dis + b_ref[...])
    ids = lax.broadcasted_iota(jnp.int32, (G, BM), 0)
    oh = (ids == jnp.reshape(bt_ref[...], (1, BM))).astype(jnp.float32)
    sums[...] += jnp.dot(oh, h, preferred_element_type=jnp.float32)
    counts[...] += jnp.broadcast_to(
        jnp.sum(oh, axis=1, keepdims=True), counts.shape)

    @pl.when(i == pl.num_programs(0) - 1)
    def _():
        hg = sums[...] / jnp.maximum(counts[:, :1], 1.0)
        mu_ref[...] = jnp.dot(hg, wmu_ref[...],
                              preferred_element_type=jnp.float32) + bmu_ref[...]
        lv_ref[...] = jnp.dot(hg, wlv_ref[...],
                              preferred_element_type=jnp.float32) + blv_ref[...]


def _final(acc3, deg, b2, batch2d, Wmu, bmu, Wlv, blv):
    return pl.pallas_call(
        _final_body,
        grid=(NP // BM,),
        in_specs=[
            pl.BlockSpec((2, BM, HH), lambda i: (0, i, 0)),
            pl.BlockSpec((2, BM, 16), lambda i: (0, i, 0)),
            pl.BlockSpec((1, HD), lambda i: (0, 0)),
            pl.BlockSpec((BM, 1), lambda i: (i, 0)),
            pl.BlockSpec((HD, L), lambda i: (0, 0)),
            pl.BlockSpec((1, L), lambda i: (0, 0)),
            pl.BlockSpec((HD, L), lambda i: (0, 0)),
            pl.BlockSpec((1, L), lambda i: (0, 0)),
        ],
        out_specs=[
            pl.BlockSpec((G, L), lambda i: (0, 0)),
            pl.BlockSpec((G, L), lambda i: (0, 0)),
        ],
        out_shape=[
            jax.ShapeDtypeStruct((G, L), jnp.float32),
            jax.ShapeDtypeStruct((G, L), jnp.float32),
        ],
        scratch_shapes=[
            pltpu.VMEM((G, HD), jnp.float32),
            pltpu.VMEM((G, 128), jnp.float32),
        ],
    )(acc3, deg, b2, batch2d, Wmu, bmu, Wlv, blv)


def kernel(x, edge_index, batch, W1, b1, W2, b2, Wmu, bmu, Wlv, blv):
    x_pad = jnp.zeros((NP, D), jnp.float32).at[:N].set(x)
    padi = jnp.full((EP - E,), N, jnp.int32)
    srcp = jnp.concatenate([edge_index[0], padi])
    dstp = jnp.concatenate([edge_index[1], padi])
    batch2d = jnp.concatenate(
        [batch, jnp.full((NP - N,), G, jnp.int32)])[:, None]
    initc = jnp.concatenate([jnp.ones((C, 16), jnp.float32),
                             jnp.zeros((C, 16), jnp.float32)])
    b1r = b1[None, :]
    b2r = b2[None, :]
    bmur = bmu[None, :]
    blvr = blv[None, :]

    deg = _deg_sc(dstp, initc).reshape(2, NP, 16)
    y1 = _mm1(x_pad, W1, deg)
    acc1 = _spmm_sc(y1, srcp, dstp)
    y2 = _mm2(acc1.reshape(2, NP, HH), W2, deg, b1r)
    acc2 = _spmm_sc(y2.reshape(2 * NP, HH), srcp, dstp)
    mu, lv = _final(acc2.reshape(2, NP, HH), deg, b2r, batch2d,
                    Wmu, bmur, Wlv, blvr)
    return (mu, lv)


# speedup vs baseline: 1.5563x; 1.2087x over previous
import functools

import jax
import jax.numpy as jnp
from jax import lax
from jax.experimental import pallas as pl
from jax.experimental.pallas import tpu as pltpu
from jax.experimental.pallas import tpu_sc as plsc

NC = 2
NS = 16
LN = 16

N = 10000
E = 160000
NP = 10240
EP = 163840
C = 128
D = 256
HD = 256
HH = 128
G = 64
L = 64

_MESH = plsc.VectorSubcoreMesh(core_axis_name="c", subcore_axis_name="s")


NCHK = EP // 128 // (NC * NS)


@functools.partial(
    pl.kernel,
    mesh=_MESH,
    out_type=jax.ShapeDtypeStruct((2 * NP, 16), jnp.float32),
    scratch_types=[
        pltpu.VMEM_SHARED((NP, 16), jnp.float32),
        pltpu.VMEM((C,), jnp.int32),
        pltpu.VMEM((C,), jnp.int32),
        pltpu.VMEM((C, 16), jnp.float32),
        pltpu.VMEM((C, 16), jnp.float32),
        pltpu.SemaphoreType.DMA,
        pltpu.SemaphoreType.DMA,
    ],
)
def _deg_sc(dst_hbm, initc_hbm, deg_hbm, acc_sh, dv0, dv1, ones_v, ini_v,
            sem_i0, sem_i1):
    cid = lax.axis_index("c")
    sid = lax.axis_index("s")
    wid = cid * NS + sid
    base = wid * NCHK * C
    dvs = (dv0, dv1)
    sems = (sem_i0, sem_i1)

    def i_start(k, p):
        pltpu.async_copy(dst_hbm.at[pl.ds(base + k * C, C)], dvs[p], sems[p])

    def i_wait(k, p):
        pltpu.make_async_copy(dst_hbm.at[pl.ds(base + k * C, C)],
                              dvs[p], sems[p]).wait()

    pltpu.sync_copy(initc_hbm.at[pl.ds(0, C)], ones_v)
    pltpu.sync_copy(initc_hbm.at[pl.ds(cid * C, C)], ini_v)
    for k in range(NP // NS // C):
        pltpu.sync_copy(ini_v, acc_sh.at[pl.ds(sid * (NP // NS) + k * C, C)])
    plsc.subcore_barrier()
    i_start(0, 0)
    i_start(1, 1)

    def body(h, _):
        for j in range(2):
            k = 2 * h + j
            i_wait(k, j)
            pltpu.sync_copy(ones_v, acc_sh.at[dvs[j]], add=True)

            @pl.when(k + 2 < NCHK)
            def _():
                i_start(k + 2, j)
        return 0

    lax.fori_loop(0, NCHK // 2, body, 0)
    plsc.subcore_barrier()
    for k in range(NP // NS // C):
        off = sid * (NP // NS) + k * C
        pltpu.sync_copy(acc_sh.at[pl.ds(off, C)], ini_v)
        pltpu.sync_copy(ini_v, deg_hbm.at[pl.ds(cid * NP + off, C)])


CS = 64
KT = EP // CS // NS
CW = 64


@functools.partial(
    pl.kernel,
    mesh=_MESH,
    out_type=jax.ShapeDtypeStruct((2 * NP, HH), jnp.float32),
    scratch_types=(
        [pltpu.VMEM_SHARED((NP, HH), jnp.float32)]
        + [pltpu.VMEM((CS,), jnp.int32)] * 16
        + [pltpu.VMEM((CS, HH), jnp.float32)] * 4
        + [pltpu.VMEM((CW, HH), jnp.float32)]
        + [pltpu.SemaphoreType.DMA] * 14
    ),
)
def _spmm_sc(y_hbm, src_hbm, dst_hbm, out_hbm, acc_sh, *bufs):
    svs = bufs[0:8]
    dvs = bufs[8:16]
    rows = bufs[16:20]
    wbuf = bufs[20]
    isems = bufs[21:29]
    gsems = bufs[29:33]
    ssems = bufs[33:35]
    cid = lax.axis_index("c")
    sid = lax.axis_index("s")
    row0 = cid * NP
    base = sid * KT * CS

    def i_start(k, p):
        pltpu.async_copy(src_hbm.at[pl.ds(base + k * CS, CS)], svs[p], isems[p])
        pltpu.async_copy(dst_hbm.at[pl.ds(base + k * CS, CS)], dvs[p], isems[p])

    def i_wait(k, p):
        pltpu.make_async_copy(src_hbm.at[pl.ds(base + k * CS, CS)],
                              svs[p], isems[p]).wait()
        pltpu.make_async_copy(dst_hbm.at[pl.ds(base + k * CS, CS)],
                              dvs[p], isems[p]).wait()
        for j in range(CS // LN):
            sl = pl.ds(j * LN, LN)
            svs[p][sl] = svs[p][sl] + row0

    def g_start(p, rp):
        pltpu.async_copy(y_hbm.at[svs[p]], rows[rp], gsems[rp])

    def g_wait(p, rp):
        pltpu.make_async_copy(y_hbm.at[svs[p]], rows[rp], gsems[rp]).wait()

    def s_start(p, rp, sp):
        pltpu.async_copy(rows[rp], acc_sh.at[dvs[p]], ssems[sp], add=True)

    def s_wait(p, rp, sp):
        pltpu.make_async_copy(rows[rp], acc_sh.at[dvs[p]], ssems[sp]).wait()

    for k in range(NP // NS // CW):
        off = sid * (NP // NS) + k * CW
        pltpu.sync_copy(y_hbm.at[pl.ds(row0 + off, CW)], wbuf)
        pltpu.sync_copy(wbuf, acc_sh.at[pl.ds(off, CW)])
    plsc.subcore_barrier()

    i_start(0, 0)
    i_start(1, 1)
    i_start(2, 2)
    i_start(3, 3)
    i_wait(0, 0)
    g_start(0, 0)
    i_wait(1, 1)
    g_start(1, 1)

    def body(h, _):
        for j in range(8):
            k = 8 * h + j
            jr = j % 4
            g_wait(j, jr)

            @pl.when(k >= 2)
            def _():
                s_wait((j + 6) % 8, (j + 2) % 4, j % 2)

            @pl.when(k + 2 < KT)
            def _():
                i_wait(k + 2, (j + 2) % 8)
                g_start((j + 2) % 8, (j + 2) % 4)
            s_start(j, jr, j % 2)

            @pl.when(k + 4 < KT)
            def _():
                i_start(k + 4, (j + 4) % 8)
        return 0

    lax.fori_loop(0, KT // 8, body, 0)
    s_wait(6, 2, 0)
    s_wait(7, 3, 1)
    plsc.subcore_barrier()

    for k in range(NP // NS // CW):
        off = sid * (NP // NS) + k * CW
        pltpu.sync_copy(acc_sh.at[pl.ds(off, CW)], wbuf)
        pltpu.sync_copy(wbuf, out_hbm.at[pl.ds(row0 + off, CW)])


BM = 512


BM1 = 400


def _mm1_body(x_ref, w_ref, deg_ref, o_ref):
    dis = lax.rsqrt(deg_ref[0, :, :1] + deg_ref[1, :, :1])
    o_ref[0] = jnp.dot(x_ref[...], w_ref[...],
                       preferred_element_type=jnp.float32) * dis


def _mm1(x, W1, deg):
    return pl.pallas_call(
        _mm1_body,
        grid=(2, N // BM1),
        in_specs=[
            pl.BlockSpec((BM1, D), lambda c, i: (i, 0)),
            pl.BlockSpec((D, HH), lambda c, i: (0, c)),
            pl.BlockSpec((2, BM1, 16), lambda c, i: (0, i, 0)),
        ],
        out_specs=pl.BlockSpec((1, BM1, HH), lambda c, i: (c, i, 0)),
        out_shape=jax.ShapeDtypeStruct((2, NP, HH), jnp.float32),
    )(x, W1, deg)


def _mm2_body(a_ref, w_ref, deg_ref, b_ref, o_ref):
    dis = lax.rsqrt(deg_ref[0, :, :1] + deg_ref[1, :, :1])
    h = jnp.concatenate([a_ref[0], a_ref[1]], axis=1)
    h = jax.nn.relu(h * dis + b_ref[...])
    y = jnp.dot(h, w_ref[...], preferred_element_type=jnp.float32) * dis
    o_ref[0] = y[:, :HH]
    o_ref[1] = y[:, HH:]


def _mm2(acc3, W2, deg, b1):
    return pl.pallas_call(
        _mm2_body,
        grid=(NP // BM,),
        in_specs=[
            pl.BlockSpec((2, BM, HH), lambda i: (0, i, 0)),
            pl.BlockSpec((HD, HD), lambda i: (0, 0)),
            pl.BlockSpec((2, BM, 16), lambda i: (0, i, 0)),
            pl.BlockSpec((1, HD), lambda i: (0, 0)),
        ],
        out_specs=pl.BlockSpec((2, BM, HH), lambda i: (0, i, 0)),
        out_shape=jax.ShapeDtypeStruct((2, NP, HH), jnp.float32),
    )(acc3, W2, deg, b1)


def _final_body(a_ref, deg_ref, b_ref, bt_ref, wmu_ref, bmu_ref, wlv_ref,
                blv_ref, mu_ref, lv_ref, sums, counts):
    i = pl.program_id(0)

    @pl.when(i == 0)
    def _():
        sums[...] = jnp.zeros_like(sums)
        counts[...] = jnp.zeros_like(counts)

    dis = lax.rsqrt(deg_ref[0, :, :1] + deg_ref[1, :, :1])
    h = jnp.concatenate([a_ref[0], a_ref[1]], axis=1)
    h = jax.nn.relu(h * dis + b_ref[...])
    h = jnp.where(bt_ref[...] < G, h, 0.0)
    ids = lax.broadcasted_iota(jnp.int32, (G, BM), 0)
    oh = (ids == jnp.reshape(bt_ref[...], (1, BM))).astype(jnp.float32)
    sums[...] += jnp.dot(oh, h, preferred_element_type=jnp.float32)
    counts[...] += jnp.broadcast_to(
        jnp.sum(oh, axis=1, keepdims=True), counts.shape)

    @pl.when(i == pl.num_programs(0) - 1)
    def _():
        hg = sums[...] / jnp.maximum(counts[:, :1], 1.0)
        mu_ref[...] = jnp.dot(hg, wmu_ref[...],
                              preferred_element_type=jnp.float32) + bmu_ref[...]
        lv_ref[...] = jnp.dot(hg, wlv_ref[...],
                              preferred_element_type=jnp.float32) + blv_ref[...]


def _final(acc3, deg, b2, batch2d, Wmu, bmu, Wlv, blv):
    return pl.pallas_call(
        _final_body,
        grid=(NP // BM,),
        in_specs=[
            pl.BlockSpec((2, BM, HH), lambda i: (0, i, 0)),
            pl.BlockSpec((2, BM, 16), lambda i: (0, i, 0)),
            pl.BlockSpec((1, HD), lambda i: (0, 0)),
            pl.BlockSpec((BM, 1), lambda i: (i, 0)),
            pl.BlockSpec((HD, L), lambda i: (0, 0)),
            pl.BlockSpec((1, L), lambda i: (0, 0)),
            pl.BlockSpec((HD, L), lambda i: (0, 0)),
            pl.BlockSpec((1, L), lambda i: (0, 0)),
        ],
        out_specs=[
            pl.BlockSpec((G, L), lambda i: (0, 0)),
            pl.BlockSpec((G, L), lambda i: (0, 0)),
        ],
        out_shape=[
            jax.ShapeDtypeStruct((G, L), jnp.float32),
            jax.ShapeDtypeStruct((G, L), jnp.float32),
        ],
        scratch_shapes=[
            pltpu.VMEM((G, HD), jnp.float32),
            pltpu.VMEM((G, 128), jnp.float32),
        ],
    )(acc3, deg, b2, batch2d, Wmu, bmu, Wlv, blv)


def kernel(x, edge_index, batch, W1, b1, W2, b2, Wmu, bmu, Wlv, blv):
    padi = jnp.full((EP - E,), N, jnp.int32)
    srcp = jnp.concatenate([edge_index[0], padi])
    dstp = jnp.concatenate([edge_index[1], padi])
    batch2d = jnp.concatenate(
        [batch, jnp.full((NP - N,), G, jnp.int32)])[:, None]
    initc = jnp.concatenate([jnp.ones((C, 16), jnp.float32),
                             jnp.zeros((C, 16), jnp.float32)])
    b1r = b1[None, :]
    b2r = b2[None, :]
    bmur = bmu[None, :]
    blvr = blv[None, :]

    deg = _deg_sc(dstp, initc).reshape(2, NP, 16)
    y1 = _mm1(x, W1, deg).reshape(2 * NP, HH)
    acc1 = _spmm_sc(y1, srcp, dstp)
    y2 = _mm2(acc1.reshape(2, NP, HH), W2, deg, b1r)
    acc2 = _spmm_sc(y2.reshape(2 * NP, HH), srcp, dstp)
    mu, lv = _final(acc2.reshape(2, NP, HH), deg, b2r, batch2d,
                    Wmu, bmur, Wlv, blvr)
    return (mu, lv)


# speedup vs baseline: 1.5603x; 1.0026x over previous
import functools

import jax
import jax.numpy as jnp
from jax import lax
from jax.experimental import pallas as pl
from jax.experimental.pallas import tpu as pltpu
from jax.experimental.pallas import tpu_sc as plsc

NC = 2
NS = 16
LN = 16

N = 10000
E = 160000
NP = 10240
EP = 163840
C = 128
D = 256
HD = 256
HH = 128
G = 64
L = 64

_MESH = plsc.VectorSubcoreMesh(core_axis_name="c", subcore_axis_name="s")


NCHK = EP // 128 // (NC * NS)


@functools.partial(
    pl.kernel,
    mesh=_MESH,
    out_type=jax.ShapeDtypeStruct((2 * NP, 16), jnp.float32),
    scratch_types=[
        pltpu.VMEM_SHARED((NP, 16), jnp.float32),
        pltpu.VMEM((C,), jnp.int32),
        pltpu.VMEM((C,), jnp.int32),
        pltpu.VMEM((C,), jnp.int32),
        pltpu.VMEM((C,), jnp.int32),
        pltpu.VMEM((C, 16), jnp.float32),
        pltpu.VMEM((C, 16), jnp.float32),
        pltpu.SemaphoreType.DMA,
        pltpu.SemaphoreType.DMA,
        pltpu.SemaphoreType.DMA,
        pltpu.SemaphoreType.DMA,
        pltpu.SemaphoreType.DMA,
        pltpu.SemaphoreType.DMA,
    ],
)
def _deg_sc(dst_hbm, initc_hbm, deg_hbm, acc_sh, dv0, dv1, dv2, dv3,
            ones_v, ini_v, sem_i0, sem_i1, sem_i2, sem_i3, sem_s0, sem_s1):
    cid = lax.axis_index("c")
    sid = lax.axis_index("s")
    wid = cid * NS + sid
    base = wid * NCHK * C
    dvs = (dv0, dv1, dv2, dv3)
    sems = (sem_i0, sem_i1, sem_i2, sem_i3)
    ssems = (sem_s0, sem_s1)

    def i_start(k, p):
        pltpu.async_copy(dst_hbm.at[pl.ds(base + k * C, C)], dvs[p], sems[p])

    def i_wait(k, p):
        pltpu.make_async_copy(dst_hbm.at[pl.ds(base + k * C, C)],
                              dvs[p], sems[p]).wait()

    def s_start(p, sp):
        pltpu.async_copy(ones_v, acc_sh.at[dvs[p]], ssems[sp], add=True)

    def s_wait(p, sp):
        pltpu.make_async_copy(ones_v, acc_sh.at[dvs[p]], ssems[sp]).wait()

    pltpu.sync_copy(initc_hbm.at[pl.ds(0, C)], ones_v)
    pltpu.sync_copy(initc_hbm.at[pl.ds(cid * C, C)], ini_v)
    for k in range(NP // NS // C):
        pltpu.sync_copy(ini_v, acc_sh.at[pl.ds(sid * (NP // NS) + k * C, C)])
    plsc.subcore_barrier()
    i_start(0, 0)
    i_start(1, 1)

    def body(h, _):
        for j in range(4):
            k = 4 * h + j
            i_wait(k, j)

            @pl.when(k >= 2)
            def _():
                s_wait((j + 2) % 4, j % 2)
            s_start(j, j % 2)

            @pl.when(k + 2 < NCHK)
            def _():
                i_start(k + 2, (j + 2) % 4)
        return 0

    lax.fori_loop(0, NCHK // 4, body, 0)
    s_wait(2, 0)
    s_wait(3, 1)
    plsc.subcore_barrier()
    for k in range(NP // NS // C):
        off = sid * (NP // NS) + k * C
        pltpu.sync_copy(acc_sh.at[pl.ds(off, C)], ini_v)
        pltpu.sync_copy(ini_v, deg_hbm.at[pl.ds(cid * NP + off, C)])


CS = 64
KT = EP // CS // NS
CW = 64


@functools.partial(
    pl.kernel,
    mesh=_MESH,
    out_type=jax.ShapeDtypeStruct((2 * NP, HH), jnp.float32),
    scratch_types=(
        [pltpu.VMEM_SHARED((NP, HH), jnp.float32)]
        + [pltpu.VMEM((CS,), jnp.int32)] * 16
        + [pltpu.VMEM((CS, HH), jnp.float32)] * 4
        + [pltpu.VMEM((CW, HH), jnp.float32)]
        + [pltpu.SemaphoreType.DMA] * 14
    ),
)
def _spmm_sc(y_hbm, src_hbm, dst_hbm, out_hbm, acc_sh, *bufs):
    svs = bufs[0:8]
    dvs = bufs[8:16]
    rows = bufs[16:20]
    wbuf = bufs[20]
    isems = bufs[21:29]
    gsems = bufs[29:33]
    ssems = bufs[33:35]
    cid = lax.axis_index("c")
    sid = lax.axis_index("s")
    row0 = cid * NP
    base = sid * KT * CS

    def i_start(k, p):
        pltpu.async_copy(src_hbm.at[pl.ds(base + k * CS, CS)], svs[p], isems[p])
        pltpu.async_copy(dst_hbm.at[pl.ds(base + k * CS, CS)], dvs[p], isems[p])

    def i_wait(k, p):
        pltpu.make_async_copy(src_hbm.at[pl.ds(base + k * CS, CS)],
                              svs[p], isems[p]).wait()
        pltpu.make_async_copy(dst_hbm.at[pl.ds(base + k * CS, CS)],
                              dvs[p], isems[p]).wait()
        for j in range(CS // LN):
            sl = pl.ds(j * LN, LN)
            svs[p][sl] = svs[p][sl] + row0

    def g_start(p, rp):
        pltpu.async_copy(y_hbm.at[svs[p]], rows[rp], gsems[rp])

    def g_wait(p, rp):
        pltpu.make_async_copy(y_hbm.at[svs[p]], rows[rp], gsems[rp]).wait()

    def s_start(p, rp, sp):
        pltpu.async_copy(rows[rp], acc_sh.at[dvs[p]], ssems[sp], add=True)

    def s_wait(p, rp, sp):
        pltpu.make_async_copy(rows[rp], acc_sh.at[dvs[p]], ssems[sp]).wait()

    for k in range(NP // NS // CW):
        off = sid * (NP // NS) + k * CW
        pltpu.sync_copy(y_hbm.at[pl.ds(row0 + off, CW)], wbuf)
        pltpu.sync_copy(wbuf, acc_sh.at[pl.ds(off, CW)])
    plsc.subcore_barrier()

    i_start(0, 0)
    i_start(1, 1)
    i_start(2, 2)
    i_start(3, 3)
    i_wait(0, 0)
    g_start(0, 0)
    i_wait(1, 1)
    g_start(1, 1)

    def body(h, _):
        for j in range(8):
            k = 8 * h + j
            jr = j % 4
            g_wait(j, jr)

            @pl.when(k >= 2)
            def _():
                s_wait((j + 6) % 8, (j + 2) % 4, j % 2)

            @pl.when(k + 2 < KT)
            def _():
                i_wait(k + 2, (j + 2) % 8)
                g_start((j + 2) % 8, (j + 2) % 4)
            s_start(j, jr, j % 2)

            @pl.when(k + 4 < KT)
            def _():
                i_start(k + 4, (j + 4) % 8)
        return 0

    lax.fori_loop(0, KT // 8, body, 0)
    s_wait(6, 2, 0)
    s_wait(7, 3, 1)
    plsc.subcore_barrier()

    for k in range(NP // NS // CW):
        off = sid * (NP // NS) + k * CW
        pltpu.sync_copy(acc_sh.at[pl.ds(off, CW)], wbuf)
        pltpu.sync_copy(wbuf, out_hbm.at[pl.ds(row0 + off, CW)])


BM = 512


BM1 = 400


def _mm1_body(x_ref, w_ref, deg_ref, o_ref):
    dis = lax.rsqrt(deg_ref[0, :, :1] + deg_ref[1, :, :1])
    o_ref[0] = jnp.dot(x_ref[...], w_ref[...],
                       preferred_element_type=jnp.float32) * dis


def _mm1(x, W1, deg):
    return pl.pallas_call(
        _mm1_body,
        grid=(2, N // BM1),
        in_specs=[
            pl.BlockSpec((BM1, D), lambda c, i: (i, 0)),
            pl.BlockSpec((D, HH), lambda c, i: (0, c)),
            pl.BlockSpec((2, BM1, 16), lambda c, i: (0, i, 0)),
        ],
        out_specs=pl.BlockSpec((1, BM1, HH), lambda c, i: (c, i, 0)),
        out_shape=jax.ShapeDtypeStruct((2, NP, HH), jnp.float32),
    )(x, W1, deg)


def _mm2_body(a_ref, w_ref, deg_ref, b_ref, o_ref):
    dis = lax.rsqrt(deg_ref[0, :, :1] + deg_ref[1, :, :1])
    h = jnp.concatenate([a_ref[0], a_ref[1]], axis=1)
    h = jax.nn.relu(h * dis + b_ref[...])
    y = jnp.dot(h, w_ref[...], preferred_element_type=jnp.float32) * dis
    o_ref[0] = y[:, :HH]
    o_ref[1] = y[:, HH:]


def _mm2(acc3, W2, deg, b1):
    return pl.pallas_call(
        _mm2_body,
        grid=(NP // BM,),
        in_specs=[
            pl.BlockSpec((2, BM, HH), lambda i: (0, i, 0)),
            pl.BlockSpec((HD, HD), lambda i: (0, 0)),
            pl.BlockSpec((2, BM, 16), lambda i: (0, i, 0)),
            pl.BlockSpec((1, HD), lambda i: (0, 0)),
        ],
        out_specs=pl.BlockSpec((2, BM, HH), lambda i: (0, i, 0)),
        out_shape=jax.ShapeDtypeStruct((2, NP, HH), jnp.float32),
    )(acc3, W2, deg, b1)


def _final_body(a_ref, deg_ref, b_ref, bt_ref, wmu_ref, bmu_ref, wlv_ref,
                blv_ref, mu_ref, lv_ref, sums, counts):
    i = pl.program_id(0)

    @pl.when(i == 0)
    def _():
        sums[...] = jnp.zeros_like(sums)
        counts[...] = jnp.zeros_like(counts)

    dis = lax.rsqrt(deg_ref[0, :, :1] + deg_ref[1, :, :1])
    h = jnp.concatenate([a_ref[0], a_ref[1]], axis=1)
    h = jax.nn.relu(h * dis + b_ref[...])
    h = jnp.where(bt_ref[...] < G, h, 0.0)
    ids = lax.broadcasted_iota(jnp.int32, (G, BM), 0)
    oh = (ids == jnp.reshape(bt_ref[...], (1, BM))).astype(jnp.float32)
    sums[...] += jnp.dot(oh, h, preferred_element_type=jnp.float32)
    counts[...] += jnp.broadcast_to(
        jnp.sum(oh, axis=1, keepdims=True), counts.shape)

    @pl.when(i == pl.num_programs(0) - 1)
    def _():
        hg = sums[...] / jnp.maximum(counts[:, :1], 1.0)
        mu_ref[...] = jnp.dot(hg, wmu_ref[...],
                              preferred_element_type=jnp.float32) + bmu_ref[...]
        lv_ref[...] = jnp.dot(hg, wlv_ref[...],
                              preferred_element_type=jnp.float32) + blv_ref[...]


def _final(acc3, deg, b2, batch2d, Wmu, bmu, Wlv, blv):
    return pl.pallas_call(
        _final_body,
        grid=(NP // BM,),
        in_specs=[
            pl.BlockSpec((2, BM, HH), lambda i: (0, i, 0)),
            pl.BlockSpec((2, BM, 16), lambda i: (0, i, 0)),
            pl.BlockSpec((1, HD), lambda i: (0, 0)),
            pl.BlockSpec((BM, 1), lambda i: (i, 0)),
            pl.BlockSpec((HD, L), lambda i: (0, 0)),
            pl.BlockSpec((1, L), lambda i: (0, 0)),
            pl.BlockSpec((HD, L), lambda i: (0, 0)),
            pl.BlockSpec((1, L), lambda i: (0, 0)),
        ],
        out_specs=[
            pl.BlockSpec((G, L), lambda i: (0, 0)),
            pl.BlockSpec((G, L), lambda i: (0, 0)),
        ],
        out_shape=[
            jax.ShapeDtypeStruct((G, L), jnp.float32),
            jax.ShapeDtypeStruct((G, L), jnp.float32),
        ],
        scratch_shapes=[
            pltpu.VMEM((G, HD), jnp.float32),
            pltpu.VMEM((G, 128), jnp.float32),
        ],
    )(acc3, deg, b2, batch2d, Wmu, bmu, Wlv, blv)


def kernel(x, edge_index, batch, W1, b1, W2, b2, Wmu, bmu, Wlv, blv):
    padi = jnp.full((EP - E,), N, jnp.int32)
    srcp = jnp.concatenate([edge_index[0], padi])
    dstp = jnp.concatenate([edge_index[1], padi])
    batch2d = jnp.concatenate(
        [batch, jnp.full((NP - N,), G, jnp.int32)])[:, None]
    initc = jnp.concatenate([jnp.ones((C, 16), jnp.float32),
                             jnp.zeros((C, 16), jnp.float32)])
    b1r = b1[None, :]
    b2r = b2[None, :]
    bmur = bmu[None, :]
    blvr = blv[None, :]

    deg = _deg_sc(dstp, initc).reshape(2, NP, 16)
    y1 = _mm1(x, W1, deg).reshape(2 * NP, HH)
    acc1 = _spmm_sc(y1, srcp, dstp)
    y2 = _mm2(acc1.reshape(2, NP, HH), W2, deg, b1r)
    acc2 = _spmm_sc(y2.reshape(2 * NP, HH), srcp, dstp)
    mu, lv = _final(acc2.reshape(2, NP, HH), deg, b2r, batch2d,
                    Wmu, bmur, Wlv, blvr)
    return (mu, lv)
